# Initial kernel scaffold; baseline (speedup 1.0000x reference)
import jax
import jax.numpy as jnp
from jax.experimental import pallas as pl


def kernel(user_bin, user_pref, user_id):
    raise NotImplementedError("write your pallas kernel here")


# speedup vs baseline: 7.9925x; 7.9925x over previous
import functools

import jax
import jax.numpy as jnp
from jax import lax
from jax.experimental import pallas as pl
from jax.experimental.pallas import tpu as pltpu
from jax.experimental.pallas import tpu_sc as plsc

K_NEIGHBORS = 300
EPS = 1e-6
BLK_B = 128


def _sc_gather(table, idx):
    B = idx.shape[0]
    D = table.shape[1]
    info = plsc.get_sparse_core_info()
    nw = info.num_cores * info.num_subcores
    bw = B // nw
    mesh = plsc.VectorSubcoreMesh(core_axis_name="c", subcore_axis_name="s")

    @functools.partial(
        pl.kernel,
        mesh=mesh,
        out_type=jax.ShapeDtypeStruct((B, D), table.dtype),
        scratch_types=[
            pltpu.VMEM((bw,), jnp.int32),
            pltpu.VMEM((bw, D), table.dtype),
            pltpu.SemaphoreType.DMA,
        ],
    )
    def k(table_hbm, idx_hbm, out_hbm, idx_v, rows_v, sem):
        wid = lax.axis_index("s") * info.num_cores + lax.axis_index("c")
        base = wid * bw
        pltpu.sync_copy(idx_hbm.at[pl.ds(base, bw)], idx_v)
        pltpu.async_copy(table_hbm.at[idx_v], rows_v, sem).wait()
        pltpu.sync_copy(rows_v, out_hbm.at[pl.ds(base, bw)])

    return k(table, idx)


def _tc_body(q_ref, ub_ref, up_ref, o_ref, ubb_ref, nu_ref):
    f32 = jnp.float32
    bf16 = jnp.bfloat16
    U = ub_ref.shape[0]

    @pl.when(pl.program_id(0) == 0)
    def _():
        ubb_ref[...] = ub_ref[...].astype(bf16)
        ones8 = jnp.ones((8, ub_ref.shape[1]), bf16)
        nu_ref[...] = lax.dot_general(ones8, ubb_ref[...],
                                      (((1,), (1,)), ((), ())),
                                      preferred_element_type=f32)

    qb = q_ref[...]
    ubb = ubb_ref[...]
    qbb = qb.astype(bf16)
    dots = lax.dot_general(qbb, ubb, (((1,), (1,)), ((), ())),
                           preferred_element_type=f32)
    nu = nu_ref[0:1]
    nq = jnp.sum(qb, axis=1, keepdims=True)
    denom = jnp.sqrt(nq) * jnp.sqrt(nu) + EPS
    sim = dots / denom
    si = lax.bitcast_convert_type(sim, jnp.int32)

    onescol = jnp.ones((U, 8), bf16)

    def _count_ge(thresh):
        m = jnp.where(si >= thresh, 1.0, 0.0).astype(bf16)
        c = lax.dot_general(m, onescol, (((1,), (0,)), ((), ())),
                            preferred_element_type=f32)
        return c[:, 0:1]

    def bs_body(i, cur):
        cand = cur | lax.shift_right_logical(jnp.int32(2 ** 30), i)
        return jnp.where(_count_ge(cand) >= K_NEIGHBORS, cand, cur)

    t = lax.fori_loop(0, 31, bs_body,
                      jnp.zeros((qb.shape[0], 1), jnp.int32))
    gt = si > t
    eq = si == t
    ng = _count_ge(t + 1)
    r = jnp.where(eq, 1.0, 0.0)
    lane = lax.broadcasted_iota(jnp.int32, r.shape, 1)
    s = 1
    while s < r.shape[1]:
        r = r + jnp.where(lane >= s, pltpu.roll(r, s, axis=1), 0.0)
        s *= 2
    sel = jnp.logical_or(gt, jnp.logical_and(eq, r <= (K_NEIGHBORS - ng)))
    w = jnp.where(sel, sim, 0.0)
    o_ref[...] = lax.dot_general(w, up_ref[...], (((1,), (0,)), ((), ())),
                                 preferred_element_type=f32)


def _tc_main(q, ub_p, up):
    B = q.shape[0]
    U, IP = ub_p.shape
    I = up.shape[1]
    return pl.pallas_call(
        _tc_body,
        grid=(B // BLK_B,),
        in_specs=[
            pl.BlockSpec((BLK_B, IP), lambda i: (i, 0)),
            pl.BlockSpec((U, IP), lambda i: (0, 0)),
            pl.BlockSpec((U, I), lambda i: (0, 0)),
        ],
        out_specs=pl.BlockSpec((BLK_B, I), lambda i: (i, 0)),
        out_shape=jax.ShapeDtypeStruct((B, I), jnp.float32),
        scratch_shapes=[
            pltpu.VMEM((U, IP), jnp.bfloat16),
            pltpu.VMEM((8, U), jnp.float32),
        ],
    )(q, ub_p, up)


def kernel(user_bin, user_pref, user_id):
    U, I = user_bin.shape
    IP = 1024
    ub_p = jnp.pad(user_bin, ((0, 0), (0, IP - I)))
    q = _sc_gather(ub_p, user_id.astype(jnp.int32))
    return _tc_main(q, ub_p, user_pref)


# speedup vs baseline: 9.9524x; 1.2452x over previous
import functools

import jax
import jax.numpy as jnp
from jax import lax
from jax.experimental import pallas as pl
from jax.experimental.pallas import tpu as pltpu
from jax.experimental.pallas import tpu_sc as plsc

K_NEIGHBORS = 300
EPS = 1e-6
BLK_B = 128


def _sc_gather(table, idx):
    B = idx.shape[0]
    D = table.shape[1]
    info = plsc.get_sparse_core_info()
    nw = info.num_cores * info.num_subcores
    bw = B // nw
    mesh = plsc.VectorSubcoreMesh(core_axis_name="c", subcore_axis_name="s")

    @functools.partial(
        pl.kernel,
        mesh=mesh,
        out_type=jax.ShapeDtypeStruct((B, D), table.dtype),
        scratch_types=[
            pltpu.VMEM((bw,), jnp.int32),
            pltpu.VMEM((bw, D), table.dtype),
            pltpu.SemaphoreType.DMA,
        ],
    )
    def k(table_hbm, idx_hbm, out_hbm, idx_v, rows_v, sem):
        wid = lax.axis_index("s") * info.num_cores + lax.axis_index("c")
        base = wid * bw
        pltpu.sync_copy(idx_hbm.at[pl.ds(base, bw)], idx_v)
        pltpu.async_copy(table_hbm.at[idx_v], rows_v, sem).wait()
        pltpu.sync_copy(rows_v, out_hbm.at[pl.ds(base, bw)])

    return k(table, idx)


def _tc_body(q_ref, ub_ref, up_ref, o_ref, ubb_ref, nu_ref):
    f32 = jnp.float32
    bf16 = jnp.bfloat16
    U = ub_ref.shape[0]

    @pl.when(pl.program_id(0) == 0)
    def _():
        ubb_ref[...] = ub_ref[...].astype(bf16)
        ones8 = jnp.ones((8, ub_ref.shape[1]), bf16)
        nu_ref[...] = lax.dot_general(ones8, ubb_ref[...],
                                      (((1,), (1,)), ((), ())),
                                      preferred_element_type=f32)

    qb = q_ref[...]
    ubb = ubb_ref[...]
    qbb = qb.astype(bf16)
    dots = lax.dot_general(qbb, ubb, (((1,), (1,)), ((), ())),
                           preferred_element_type=f32)
    nu = nu_ref[0:1]
    nq = jnp.sum(qb, axis=1, keepdims=True)
    denom = jnp.sqrt(nq) * jnp.sqrt(nu) + EPS
    sim = dots / denom
    si = lax.bitcast_convert_type(sim, jnp.int32)

    def _count_ge(thresh):
        return jnp.sum(jnp.where(si >= thresh, 1.0, 0.0),
                       axis=1, keepdims=True)

    def bs_body(i, cur):
        cand = cur | lax.shift_right_logical(jnp.int32(2 ** 30), i)
        return jnp.where(_count_ge(cand) >= K_NEIGHBORS, cand, cur)

    t = lax.fori_loop(1, 31, bs_body,
                      jnp.zeros((qb.shape[0], 1), jnp.int32))
    gt = si > t
    eq = si == t
    ng = _count_ge(t + 1)
    r = jnp.where(eq, 1.0, 0.0)
    lane = lax.broadcasted_iota(jnp.int32, r.shape, 1)
    s = 1
    while s < r.shape[1]:
        r = r + jnp.where(lane >= s, pltpu.roll(r, s, axis=1), 0.0)
        s *= 2
    sel = jnp.logical_or(gt, jnp.logical_and(eq, r <= (K_NEIGHBORS - ng)))
    w = jnp.where(sel, sim, 0.0)
    o_ref[...] = lax.dot_general(w, up_ref[...], (((1,), (0,)), ((), ())),
                                 preferred_element_type=f32)


def _tc_main(q, ub_p, up):
    B = q.shape[0]
    U, IP = ub_p.shape
    I = up.shape[1]
    return pl.pallas_call(
        _tc_body,
        grid=(B // BLK_B,),
        in_specs=[
            pl.BlockSpec((BLK_B, IP), lambda i: (i, 0)),
            pl.BlockSpec((U, IP), lambda i: (0, 0)),
            pl.BlockSpec((U, I), lambda i: (0, 0)),
        ],
        out_specs=pl.BlockSpec((BLK_B, I), lambda i: (i, 0)),
        out_shape=jax.ShapeDtypeStruct((B, I), jnp.float32),
        scratch_shapes=[
            pltpu.VMEM((U, IP), jnp.bfloat16),
            pltpu.VMEM((8, U), jnp.float32),
        ],
    )(q, ub_p, up)


def kernel(user_bin, user_pref, user_id):
    U, I = user_bin.shape
    IP = 1024
    ub_p = jnp.pad(user_bin, ((0, 0), (0, IP - I)))
    q = _sc_gather(ub_p, user_id.astype(jnp.int32))
    return _tc_main(q, ub_p, user_pref)
